# baseline (speedup 1.0000x reference)
import jax
import jax.numpy as jnp
from jax.experimental import pallas as pl
from jax.experimental.pallas import tpu as pltpu

_B = 2
_D = 64
_L2_LO, _L2_N = 256, 128
_L3_LO, _L3_N = 384, 10
_EPB = _L2_N * _L3_N

_W0 = _EPB
_B0 = _W0 + 5 * _D
_N0 = _B0 + 8
_N3 = _N0 + _L2_N
_ROWS = _N3 + _L3_N


def _bd(m):
    z = jnp.zeros_like(m)
    return jnp.concatenate(
        [jnp.concatenate([m, z], axis=1), jnp.concatenate([z, m], axis=1)],
        axis=0)


def _gnn_kernel(p_ref, out_ref):
    d = _D
    f32 = jnp.float32

    def mm(a, b):
        return jnp.dot(a, b, preferred_element_type=f32)

    Wn = p_ref[_W0:_W0 + d, 0:d]
    We = p_ref[_W0:_W0 + d, d:2 * d]
    Wm1 = p_ref[_W0 + d:_W0 + 2 * d, 0:d]
    Wm2 = p_ref[_W0 + d:_W0 + 2 * d, d:2 * d]
    Wm3 = p_ref[_W0 + 2 * d:_W0 + 3 * d, 0:d]
    Wu1 = p_ref[_W0 + 2 * d:_W0 + 3 * d, d:2 * d]
    Wu2 = p_ref[_W0 + 3 * d:_W0 + 4 * d, 0:d]
    W1 = p_ref[_W0 + 3 * d:_W0 + 4 * d, d:2 * d]
    W2 = p_ref[_W0 + 4 * d:_W0 + 5 * d, 0:d]
    W3 = p_ref[_W0 + 4 * d:_W0 + 5 * d, d:d + _L3_N]
    bn = p_ref[_B0:_B0 + 1, 0:d]
    be = p_ref[_B0:_B0 + 1, d:2 * d]
    bm = p_ref[_B0 + 1:_B0 + 2, 0:d]
    bu = p_ref[_B0 + 1:_B0 + 2, d:2 * d]
    b1 = p_ref[_B0 + 2:_B0 + 3, 0:d]
    b2 = p_ref[_B0 + 2:_B0 + 3, d:2 * d]
    b3 = p_ref[_B0 + 3:_B0 + 4, 0:_L3_N]
    e_pair = p_ref[0:_EPB, :]
    n2p = p_ref[_N0:_N3, :]
    n3p = p_ref[_N3:_ROWS, :]

    def two(v):
        return jnp.concatenate([v, v], axis=1)

    A1 = mm(Wn, Wm1)
    A2 = mm(Wn, Wm2)
    A3 = mm(We, Wm3)
    const = two(mm(bn, Wm1) + mm(bn, Wm2) + mm(be, Wm3) + bm)

    r = jax.lax.broadcasted_iota(jnp.int32, (_EPB, 1), 0)
    a_id = r // _L3_N
    j_id = r - a_id * _L3_N
    Q = (jax.lax.broadcasted_iota(jnp.int32, (1, _L2_N), 1)
         == a_id).astype(f32)
    P = (jax.lax.broadcasted_iota(jnp.int32, (1, _L3_N), 1)
         == j_id).astype(f32)

    ea = mm(e_pair, _bd(A3))
    xs2 = mm(n2p, _bd(A1))
    xd3 = mm(n3p, _bd(A2))
    msg = jax.nn.relu(ea + mm(Q, xs2) + mm(P, xd3) + const)
    agg = jax.lax.dot_general(P, msg, (((0,), (0,)), ((), ())),
                              preferred_element_type=f32)

    x3 = mm(n3p, _bd(Wn)) + two(bn)
    node = jax.nn.relu(mm(x3, _bd(Wu1)) + mm(agg, _bd(Wu2)) + two(bu))
    gf = jnp.mean(node, axis=0, keepdims=True)
    h = jax.nn.relu(mm(gf, _bd(W1)) + two(b1))
    h = jax.nn.relu(mm(h, _bd(W2)) + two(b2))
    out20 = mm(h, _bd(W3)) + two(b3)
    out_ref[...] = jnp.concatenate(
        [out20[:, 0:_L3_N], out20[:, _L3_N:2 * _L3_N]], axis=0)


def kernel(inputs_nodes, inputs_edges, Wn, bn, We, be, Wm, bm, Wu, bu,
           W1, b1, W2, b2, W3, b3):
    d = _D
    e_blk = jax.lax.slice(inputs_edges,
                          (0, _L2_LO, _L3_LO, 0),
                          (_B, _L2_LO + _L2_N, _L3_LO + _L3_N, _D))
    e_pair = jnp.concatenate([e_blk[0].reshape(_EPB, d),
                              e_blk[1].reshape(_EPB, d)], axis=1)
    wp = jnp.concatenate([
        jnp.concatenate([Wn, We], axis=1),
        jnp.concatenate([Wm[0:d], Wm[d:2 * d]], axis=1),
        jnp.concatenate([Wm[2 * d:3 * d], Wu[0:d]], axis=1),
        jnp.concatenate([Wu[d:2 * d], W1], axis=1),
        jnp.concatenate([W2, jnp.pad(W3, ((0, 0), (0, d - _L3_N)))],
                        axis=1),
    ], axis=0)
    bp = jnp.concatenate([
        jnp.concatenate([bn, be]).reshape(1, 2 * d),
        jnp.concatenate([bm, bu]).reshape(1, 2 * d),
        jnp.concatenate([b1, b2]).reshape(1, 2 * d),
        jnp.pad(b3, (0, 2 * d - _L3_N)).reshape(1, 2 * d),
        jnp.zeros((4, 2 * d), jnp.float32),
    ], axis=0)
    npair = jnp.concatenate([inputs_nodes[0, _L2_LO:_L3_LO + _L3_N, :],
                             inputs_nodes[1, _L2_LO:_L3_LO + _L3_N, :]],
                            axis=1)
    packed = jnp.concatenate([e_pair, wp, bp, npair], axis=0)

    vmem = pl.BlockSpec(memory_space=pltpu.MemorySpace.VMEM)
    return pl.pallas_call(
        _gnn_kernel,
        out_shape=jax.ShapeDtypeStruct((_B, _L3_N), jnp.float32),
        in_specs=[vmem],
        out_specs=vmem,
    )(packed)


# speedup vs baseline: 1.1787x; 1.1787x over previous
import jax
import jax.numpy as jnp
from jax.experimental import pallas as pl
from jax.experimental.pallas import tpu as pltpu

_B = 2
_D = 64
_L2_LO, _L2_N = 256, 128
_L3_LO, _L3_N = 384, 10


def _gnn_kernel(nodes_ref, e_ref, Wn_ref, bn_ref, We_ref, be_ref,
                Wm_ref, bm_ref, Wu_ref, bu_ref, W1_ref, b1_ref,
                W2_ref, b2_ref, W3_ref, b3_ref, out_ref):
    d = _D
    Wn = Wn_ref[...]
    Wm1 = Wm_ref[0:d, :]
    Wm2 = Wm_ref[d:2 * d, :]
    Wm3 = Wm_ref[2 * d:3 * d, :]
    Wu1 = Wu_ref[0:d, :]
    Wu2 = Wu_ref[d:2 * d, :]
    bn = bn_ref[...]
    be = be_ref[...]
    bm = bm_ref[...]
    We = We_ref[...]

    outs = []
    for b in range(_B):
        n2 = nodes_ref[b, pl.ds(_L2_LO, _L2_N), :]
        n3 = nodes_ref[b, pl.ds(_L3_LO, _L3_N), :]
        x2 = jnp.dot(n2, Wn, preferred_element_type=jnp.float32) + bn
        x3 = jnp.dot(n3, Wn, preferred_element_type=jnp.float32) + bn
        xs2 = jnp.dot(x2, Wm1, preferred_element_type=jnp.float32)
        xd3 = jnp.dot(x3, Wm2, preferred_element_type=jnp.float32)
        aggs = []
        for j in range(_L3_N):
            ej = e_ref[b, :, j, :]
            ew = jnp.dot(ej, We, preferred_element_type=jnp.float32) + be
            ea = jnp.dot(ew, Wm3, preferred_element_type=jnp.float32)
            m = jax.nn.relu(ea + xs2 + xd3[j:j + 1, :] + bm)
            aggs.append(jnp.sum(m, axis=0, keepdims=True))
        agg = jnp.concatenate(aggs, axis=0)
        node = jax.nn.relu(jnp.dot(x3, Wu1, preferred_element_type=jnp.float32)
                           + jnp.dot(agg, Wu2, preferred_element_type=jnp.float32)
                           + bu_ref[...])
        gf = jnp.mean(node, axis=0, keepdims=True)
        h = jax.nn.relu(jnp.dot(gf, W1_ref[...],
                                preferred_element_type=jnp.float32) + b1_ref[...])
        h = jax.nn.relu(jnp.dot(h, W2_ref[...],
                                preferred_element_type=jnp.float32) + b2_ref[...])
        outs.append(jnp.dot(h, W3_ref[...],
                            preferred_element_type=jnp.float32) + b3_ref[...])
    out_ref[...] = jnp.concatenate(outs, axis=0)


def kernel(inputs_nodes, inputs_edges, Wn, bn, We, be, Wm, bm, Wu, bu,
           W1, b1, W2, b2, W3, b3):
    e_blk = jax.lax.slice(inputs_edges,
                          (0, _L2_LO, _L3_LO, 0),
                          (_B, _L2_LO + _L2_N, _L3_LO + _L3_N, _D))
    vmem = pl.BlockSpec(memory_space=pltpu.MemorySpace.VMEM)
    return pl.pallas_call(
        _gnn_kernel,
        out_shape=jax.ShapeDtypeStruct((_B, _L3_N), jnp.float32),
        in_specs=[vmem] * 16,
        out_specs=vmem,
    )(inputs_nodes, e_blk, Wn, bn.reshape(1, _D), We, be.reshape(1, _D),
      Wm, bm.reshape(1, _D), Wu, bu.reshape(1, _D), W1, b1.reshape(1, _D),
      W2, b2.reshape(1, _D), W3, b3.reshape(1, -1))
